# baseline (speedup 1.0000x reference)
import functools

import jax
import jax.numpy as jnp
from jax import lax
from jax.experimental import pallas as pl
from jax.experimental.pallas import tpu as pltpu
from jax.experimental.pallas import tpu_sc as plsc

_N = 1600000
_NC = 2
_NS = 16
_NW = _NC * _NS
_PER_W = _N // _NW
_C = 10000
_NCHUNK = _PER_W // _C


def _sc_body(p_hbm, a_hbm, b_hbm, o_hbm, p_v, a_v, b_v, o_v):
    wid = lax.axis_index("s") * _NC + lax.axis_index("c")
    wbase = wid * _PER_W

    def compute_chunk():
        def inner(i, _):
            for j in range(5):
                s = pl.ds(i * 80 + j * 16, 16)
                o_v[s] = a_v[s] + b_v[s] * jnp.maximum(p_v[s], 0.0)
            return 0
        lax.fori_loop(0, _C // 80, inner, 0, unroll=False)

    for g in range(_NCHUNK):
        base = wbase + g * _C
        sl = pl.ds(base, _C)
        pltpu.sync_copy(p_hbm.at[sl], p_v)
        pltpu.sync_copy(a_hbm.at[sl], a_v)
        pltpu.sync_copy(b_hbm.at[sl], b_v)
        compute_chunk()
        pltpu.sync_copy(o_v, o_hbm.at[sl])


def kernel(products, A, B):
    mesh = plsc.VectorSubcoreMesh(core_axis_name="c", subcore_axis_name="s")
    run = functools.partial(
        pl.kernel,
        mesh=mesh,
        out_type=jax.ShapeDtypeStruct((_N,), jnp.float32),
        scratch_types=[
            pltpu.VMEM((_C,), jnp.float32),
            pltpu.VMEM((_C,), jnp.float32),
            pltpu.VMEM((_C,), jnp.float32),
            pltpu.VMEM((_C,), jnp.float32),
        ],
    )(_sc_body)
    return run(products, A, B)


# speedup vs baseline: 1.3429x; 1.3429x over previous
import functools

import jax
import jax.numpy as jnp
from jax import lax
from jax.experimental import pallas as pl
from jax.experimental.pallas import tpu as pltpu
from jax.experimental.pallas import tpu_sc as plsc

_N = 1600000
_NC = 2
_NS = 16
_NW = _NC * _NS
_PER_W = _N // _NW
_C = 10000
_NCHUNK = _PER_W // _C


def _sc_body(p_hbm, a_hbm, b_hbm, o_hbm,
             p0, a0, b0, o0, p1, a1, b1, o1, in_sem, out_sem):
    wid = lax.axis_index("s") * _NC + lax.axis_index("c")
    wbase = wid * _PER_W
    bufs = ((p0, a0, b0, o0), (p1, a1, b1, o1))

    pend_in = {}
    pend_out = {}

    def start_in(g):
        slot = g % 2
        pv, av, bv, _ = bufs[slot]
        sl = pl.ds(wbase + g * _C, _C)
        pend_in[slot] = [
            pltpu.async_copy(p_hbm.at[sl], pv, in_sem.at[slot]),
            pltpu.async_copy(a_hbm.at[sl], av, in_sem.at[slot]),
            pltpu.async_copy(b_hbm.at[sl], bv, in_sem.at[slot]),
        ]

    def compute_chunk(slot):
        ps, as_, bs, os_ = bufs[slot]

        def inner(i, _):
            for j in range(5):
                s = pl.ds(i * 80 + j * 16, 16)
                os_[s] = as_[s] + bs[s] * jnp.maximum(ps[s], 0.0)
            return 0
        lax.fori_loop(0, _C // 80, inner, 0, unroll=False)

    start_in(0)
    for g in range(_NCHUNK):
        slot = g % 2
        if g + 1 < _NCHUNK:
            start_in(g + 1)
        for cp in pend_in[slot]:
            cp.wait()
        if g >= 2:
            pend_out[slot].wait()
        compute_chunk(slot)
        pend_out[slot] = pltpu.async_copy(
            bufs[slot][3], o_hbm.at[pl.ds(wbase + g * _C, _C)], out_sem.at[slot])
    for slot in (0, 1) if _NCHUNK >= 2 else (0,):
        pend_out[slot].wait()


def kernel(products, A, B):
    mesh = plsc.VectorSubcoreMesh(core_axis_name="c", subcore_axis_name="s")
    run = functools.partial(
        pl.kernel,
        mesh=mesh,
        out_type=jax.ShapeDtypeStruct((_N,), jnp.float32),
        scratch_types=(
            [pltpu.VMEM((_C,), jnp.float32) for _ in range(8)]
            + [pltpu.SemaphoreType.DMA((2,)),
               pltpu.SemaphoreType.DMA((2,))]),
    )(_sc_body)
    return run(products, A, B)
